# baseline (speedup 1.0000x reference)
import functools
import jax
import jax.numpy as jnp
from jax import lax
from jax.experimental import pallas as pl
from jax.experimental.pallas import tpu as pltpu, tpu_sc as plsc

_D = 64
_DP = 128
_V = 1000000
_N = 16384
_S = 50
_SP = 56

_INFO = plsc.get_sparse_core_info()
_NC, _NS = _INFO.num_cores, _INFO.num_subcores
_NW = _NC * _NS
_MESH = plsc.VectorSubcoreMesh(core_axis_name="c", subcore_axis_name="s")


def _make_transpose(V):
  TL = 800
  n_tiles = V // TL
  NT = (n_tiles + _NW - 1) // _NW

  @functools.partial(
      pl.kernel,
      out_type=jax.ShapeDtypeStruct((V, _DP), jnp.float32),
      mesh=_MESH,
      compiler_params=pltpu.CompilerParams(
          use_tc_tiling_on_sc=False, needs_layout_passes=False),
      scratch_types=[
          pltpu.VMEM((_D, TL), jnp.float32),
          pltpu.VMEM((TL, _D), jnp.float32),
      ],
  )
  def transpose_kernel(tt_hbm, out_hbm, in_v, outt_v):
    wid = lax.axis_index("s") * _NC + lax.axis_index("c")
    kiota = lax.iota(jnp.int32, 16)

    @pl.loop(0, NT)
    def _t(t):
      tg = t * _NW + wid

      @pl.when(tg < n_tiles)
      def _():
        col0 = tg * TL
        pltpu.sync_copy(tt_hbm.at[:, pl.ds(col0, TL)], in_v)
        for k0 in range(0, _D, 16):
          kvec = kiota + k0

          @pl.loop(0, TL, unroll=8)
          def _r(r):
            rv = jnp.full((16,), 0, jnp.int32) + r
            v = plsc.load_gather(in_v, [kvec, rv])
            outt_v[r, pl.ds(k0, 16)] = v

        pltpu.sync_copy(outt_v, out_hbm.at[pl.ds(col0, TL), pl.ds(0, _D)])

  return transpose_kernel


def _make_gather(N, S, D):
  assert N % _NW == 0
  n_per_w = N // _NW
  R = 8
  NBUF = 4
  assert n_per_w % (R * NBUF) == 0
  n_chunks = n_per_w // R

  @functools.partial(
      pl.kernel,
      out_type=jax.ShapeDtypeStruct((N, _SP, _DP), jnp.float32),
      mesh=_MESH,
      compiler_params=pltpu.CompilerParams(use_tc_tiling_on_sc=False),
      scratch_types=[
          pltpu.VMEM((n_per_w, S), jnp.int32),
          pltpu.VMEM((NBUF, R, S, _D), jnp.float32),
          [pltpu.SemaphoreType.DMA] * NBUF,
          [pltpu.SemaphoreType.DMA] * NBUF,
      ],
  )
  def gather_kernel(x_hbm, table_hbm, out_hbm, idx_v, rows_v, gsems, wsems):
    wid = lax.axis_index("s") * _NC + lax.axis_index("c")
    base = wid * n_per_w
    pltpu.sync_copy(x_hbm.at[pl.ds(base, n_per_w)], idx_v)

    def start_gather(c, b):
      for i in range(R):
        pltpu.async_copy(
            table_hbm.at[idx_v.at[c * R + i]], rows_v.at[b, i], gsems[b])

    def wait_gather(b):
      for i in range(R):
        pltpu.make_async_copy(
            table_hbm.at[idx_v.at[0]], rows_v.at[b, i], gsems[b]).wait()

    def start_write(c, b):
      pltpu.async_copy(
          rows_v.at[b],
          out_hbm.at[pl.ds(base + c * R, R), pl.ds(0, S), pl.ds(0, _D)],
          wsems[b])

    def wait_write(b):
      pltpu.make_async_copy(
          rows_v.at[b],
          out_hbm.at[pl.ds(base, R), pl.ds(0, S), pl.ds(0, _D)],
          wsems[b]).wait()

    for c in range(NBUF - 1):
      start_gather(c, c)

    for j in range(NBUF):
      wait_gather(j)
      start_write(j, j)
      pf = j + NBUF - 1
      if pf >= NBUF:
        wait_write(pf % NBUF)
      start_gather(pf, pf % NBUF)

    @pl.loop(1, n_chunks // NBUF - 1)
    def _group(i):
      c0 = i * NBUF
      for j in range(NBUF):
        c = c0 + j
        wait_gather(j)
        start_write(c, j)
        wait_write((j + NBUF - 1) % NBUF)
        start_gather(c + NBUF - 1, (j + NBUF - 1) % NBUF)

    c0 = n_chunks - NBUF
    for j in range(NBUF):
      c = c0 + j
      wait_gather(j)
      start_write(c, j)
      if c + NBUF - 1 < n_chunks:
        wait_write((j + NBUF - 1) % NBUF)
        start_gather(c + NBUF - 1, (j + NBUF - 1) % NBUF)
    for j in range(NBUF):
      wait_write(j)

  return gather_kernel


_transpose = _make_transpose(_V)
_gather = _make_gather(_N, _S, _D)


def kernel(x, table):
  table_p = _transpose(table.T)
  table_v = table_p.reshape(2 * table.shape[0], _D)
  out_p = _gather(x * 2, table_v)
  return out_p[:, :_S, :_D]


# speedup vs baseline: 1.0416x; 1.0416x over previous
import functools
import jax
import jax.numpy as jnp
from jax import lax
from jax.experimental import pallas as pl
from jax.experimental.pallas import tpu as pltpu, tpu_sc as plsc

_D = 64
_DP = 128
_V = 1000000
_N = 16384
_S = 50
_SP = 56

_INFO = plsc.get_sparse_core_info()
_NC, _NS = _INFO.num_cores, _INFO.num_subcores
_NW = _NC * _NS
_MESH = plsc.VectorSubcoreMesh(core_axis_name="c", subcore_axis_name="s")


def _make_transpose(V):
  TL = 800
  n_tiles = V // TL
  NT = (n_tiles + _NW - 1) // _NW

  @functools.partial(
      pl.kernel,
      out_type=jax.ShapeDtypeStruct((V, _DP), jnp.float32),
      mesh=_MESH,
      compiler_params=pltpu.CompilerParams(
          use_tc_tiling_on_sc=False, needs_layout_passes=False),
      scratch_types=[
          pltpu.VMEM((_D, TL), jnp.float32),
          pltpu.VMEM((TL, _D), jnp.float32),
      ],
  )
  def transpose_kernel(tt_hbm, out_hbm, in_v, outt_v):
    wid = lax.axis_index("s") * _NC + lax.axis_index("c")
    kiota = lax.iota(jnp.int32, 16)

    @pl.loop(0, NT)
    def _t(t):
      tg = t * _NW + wid

      @pl.when(tg < n_tiles)
      def _():
        col0 = tg * TL
        pltpu.sync_copy(tt_hbm.at[:, pl.ds(col0, TL)], in_v)

        @pl.loop(0, TL // 16)
        def _rb(rb):
          r0 = rb * 16
          rvec = kiota + r0
          for k in range(_D):
            v = in_v[k, pl.ds(r0, 16)]
            plsc.store_scatter(
                outt_v, [rvec, jnp.full((16,), k, jnp.int32)], v)

        pltpu.sync_copy(outt_v, out_hbm.at[pl.ds(col0, TL), pl.ds(0, _D)])

  return transpose_kernel


def _make_gather(N, S, D):
  assert N % _NW == 0
  n_per_w = N // _NW
  R = 8
  NBUF = 4
  assert n_per_w % (R * NBUF) == 0
  n_chunks = n_per_w // R

  @functools.partial(
      pl.kernel,
      out_type=jax.ShapeDtypeStruct((N, _SP, _DP), jnp.float32),
      mesh=_MESH,
      compiler_params=pltpu.CompilerParams(use_tc_tiling_on_sc=False),
      scratch_types=[
          pltpu.VMEM((n_per_w, S), jnp.int32),
          pltpu.VMEM((NBUF, R, S, _D), jnp.float32),
          [pltpu.SemaphoreType.DMA] * NBUF,
          [pltpu.SemaphoreType.DMA] * NBUF,
      ],
  )
  def gather_kernel(x_hbm, table_hbm, out_hbm, idx_v, rows_v, gsems, wsems):
    wid = lax.axis_index("s") * _NC + lax.axis_index("c")
    base = wid * n_per_w
    pltpu.sync_copy(x_hbm.at[pl.ds(base, n_per_w)], idx_v)

    def start_gather(c, b):
      for i in range(R):
        pltpu.async_copy(
            table_hbm.at[idx_v.at[c * R + i]], rows_v.at[b, i], gsems[b])

    def wait_gather(b):
      for i in range(R):
        pltpu.make_async_copy(
            table_hbm.at[idx_v.at[0]], rows_v.at[b, i], gsems[b]).wait()

    def start_write(c, b):
      pltpu.async_copy(
          rows_v.at[b],
          out_hbm.at[pl.ds(base + c * R, R), pl.ds(0, S), pl.ds(0, _D)],
          wsems[b])

    def wait_write(b):
      pltpu.make_async_copy(
          rows_v.at[b],
          out_hbm.at[pl.ds(base, R), pl.ds(0, S), pl.ds(0, _D)],
          wsems[b]).wait()

    for c in range(NBUF - 1):
      start_gather(c, c)

    for j in range(NBUF):
      wait_gather(j)
      start_write(j, j)
      pf = j + NBUF - 1
      if pf >= NBUF:
        wait_write(pf % NBUF)
      start_gather(pf, pf % NBUF)

    @pl.loop(1, n_chunks // NBUF - 1)
    def _group(i):
      c0 = i * NBUF
      for j in range(NBUF):
        c = c0 + j
        wait_gather(j)
        start_write(c, j)
        wait_write((j + NBUF - 1) % NBUF)
        start_gather(c + NBUF - 1, (j + NBUF - 1) % NBUF)

    c0 = n_chunks - NBUF
    for j in range(NBUF):
      c = c0 + j
      wait_gather(j)
      start_write(c, j)
      if c + NBUF - 1 < n_chunks:
        wait_write((j + NBUF - 1) % NBUF)
        start_gather(c + NBUF - 1, (j + NBUF - 1) % NBUF)
    for j in range(NBUF):
      wait_write(j)

  return gather_kernel


_transpose = _make_transpose(_V)
_gather = _make_gather(_N, _S, _D)


def kernel(x, table):
  table_p = _transpose(table.T)
  table_v = table_p.reshape(2 * table.shape[0], _D)
  out_p = _gather(x * 2, table_v)
  return out_p[:, :_S, :_D]


# speedup vs baseline: 8.0732x; 7.7510x over previous
import functools
import jax
import jax.numpy as jnp
from jax import lax
from jax.experimental import pallas as pl
from jax.experimental.pallas import tpu as pltpu, tpu_sc as plsc

_D = 64
_DP = 128
_N = 16384
_S = 50
_SP = 56


def _make_gather(N, S, D):
  info = plsc.get_sparse_core_info()
  NC, NS = info.num_cores, info.num_subcores
  NW = NC * NS
  assert N % NW == 0
  n_per_w = N // NW
  R = 8
  NBUF = 4
  assert n_per_w % (R * NBUF) == 0
  n_chunks = n_per_w // R

  mesh = plsc.VectorSubcoreMesh(core_axis_name="c", subcore_axis_name="s")

  @functools.partial(
      pl.kernel,
      out_type=jax.ShapeDtypeStruct((N, _SP, _DP), jnp.float32),
      mesh=mesh,
      compiler_params=pltpu.CompilerParams(use_tc_tiling_on_sc=False),
      scratch_types=[
          pltpu.VMEM((n_per_w, S), jnp.int32),
          pltpu.VMEM((NBUF, R, S, _D), jnp.float32),
          [pltpu.SemaphoreType.DMA] * NBUF,
          [pltpu.SemaphoreType.DMA] * NBUF,
      ],
  )
  def gather_kernel(x_hbm, table_hbm, out_hbm, idx_v, rows_v, gsems, wsems):
    wid = lax.axis_index("s") * NC + lax.axis_index("c")
    base = wid * n_per_w
    pltpu.sync_copy(x_hbm.at[pl.ds(base, n_per_w)], idx_v)

    def start_gather(c, b):
      for i in range(R):
        pltpu.async_copy(
            table_hbm.at[idx_v.at[c * R + i]], rows_v.at[b, i], gsems[b])

    def wait_gather(b):
      for i in range(R):
        pltpu.make_async_copy(
            table_hbm.at[idx_v.at[0]], rows_v.at[b, i], gsems[b]).wait()

    def start_write(c, b):
      pltpu.async_copy(
          rows_v.at[b],
          out_hbm.at[pl.ds(base + c * R, R), pl.ds(0, S), pl.ds(0, _D)],
          wsems[b])

    def wait_write(b):
      pltpu.make_async_copy(
          rows_v.at[b],
          out_hbm.at[pl.ds(base, R), pl.ds(0, S), pl.ds(0, _D)],
          wsems[b]).wait()

    for c in range(NBUF - 1):
      start_gather(c, c)

    for j in range(NBUF):
      wait_gather(j)
      start_write(j, j)
      pf = j + NBUF - 1
      if pf >= NBUF:
        wait_write(pf % NBUF)
      start_gather(pf, pf % NBUF)

    @pl.loop(1, n_chunks // NBUF - 1)
    def _group(i):
      c0 = i * NBUF
      for j in range(NBUF):
        c = c0 + j
        wait_gather(j)
        start_write(c, j)
        wait_write((j + NBUF - 1) % NBUF)
        start_gather(c + NBUF - 1, (j + NBUF - 1) % NBUF)

    c0 = n_chunks - NBUF
    for j in range(NBUF):
      c = c0 + j
      wait_gather(j)
      start_write(c, j)
      if c + NBUF - 1 < n_chunks:
        wait_write((j + NBUF - 1) % NBUF)
        start_gather(c + NBUF - 1, (j + NBUF - 1) % NBUF)
    for j in range(NBUF):
      wait_write(j)

  return gather_kernel


_gather = _make_gather(_N, _S, _D)


def kernel(x, table):
  table_p = jnp.pad(table, ((0, 0), (0, _DP - _D)))
  table_v = table_p.reshape(2 * table.shape[0], _D)
  out_p = _gather(x * 2, table_v)
  return out_p[:, :_S, :_D]
